# baseline (speedup 1.0000x reference)
import functools

import jax
import jax.numpy as jnp
import numpy as np
from jax import lax
from jax.experimental import pallas as pl
from jax.experimental.pallas import tpu as pltpu
from jax.experimental.pallas import tpu_sc as plsc

_VOCAB_SIZES = [100003, 100019, 100043, 100057, 100069, 100103, 100109,
                100129, 100151, 100153, 100169, 100183, 100189, 100193,
                100207, 100213]
_OFFSETS = np.concatenate([[0], np.cumsum(_VOCAB_SIZES)[:-1]]).astype(np.int32)
_B, _S, _G, _D = 4, 2048, 2, 1024
_H = 16
_HD = 16
_E = _H * _HD
_K = 4
_DIL = 3
_PAD = (_K - 1) * _DIL
_EPS = 1e-6

_NW = 32
_NIDX = _B * _S * _H
_PERW = _NIDX // _NW
_CHUNK = 128
_NCHUNK = _PERW // _CHUNK


_V = sum(_VOCAB_SIZES)
_CV = 8192
_NCV = -(-_V // _CV)
_VP = _NCV * _CV


def _tc_prep(ids2):
    def body(i_ref, offs_ref, o_ref):
        x = i_ref[...] + offs_ref[...][0:1, :]
        x3 = x.reshape(_NIDX // 128, 8, _H)
        o_ref[...] = jnp.concatenate([x3[:, c, :] for c in range(8)], axis=1)

    offs8 = jnp.tile(jnp.asarray(_OFFSETS, dtype=jnp.int32)[None, :], (8, 1))
    return pl.pallas_call(
        body,
        in_specs=[pl.BlockSpec((_B * _S, _H), lambda: (0, 0)),
                  pl.BlockSpec((8, _H), lambda: (0, 0))],
        out_specs=pl.BlockSpec((_NIDX // 128, 128), lambda: (0, 0)),
        out_shape=jax.ShapeDtypeStruct((_NIDX // 128, 128), jnp.int32),
    )(ids2, offs8)


_PCH = 512
_NPCH = -(-(_V - 8) // _PCH)
_PAL = (_V - _PCH) // 8 * 8


def _sc_pack(table):
    mesh = plsc.VectorSubcoreMesh(core_axis_name="c", subcore_axis_name="s")

    @functools.partial(
        pl.kernel,
        out_type=jax.ShapeDtypeStruct((_VP // 8, 128), jnp.float32),
        mesh=mesh,
        scratch_types=[
            pltpu.VMEM((_PCH, _HD), jnp.float32),
            pltpu.VMEM((_PCH // 8, 128), jnp.float32),
        ],
    )
    def k(table_hbm, out_hbm, in_v, out_v):
        wid = lax.axis_index("s") * 2 + lax.axis_index("c")

        def chunk(i, _):
            c = i * _NW + wid
            row0 = pl.multiple_of(jnp.minimum(c * _PCH, _PAL), 64)
            pltpu.sync_copy(table_hbm.at[pl.ds(row0, _PCH)], in_v)

            def cb(j, _):
                for c8 in range(8):
                    out_v[j, pl.ds(c8 * _HD, _HD)] = in_v[8 * j + c8, :]
                return j

            lax.fori_loop(0, _PCH // 8, cb, 0)
            r8 = pl.multiple_of(row0 // 8, 8)
            pltpu.sync_copy(out_v, out_hbm.at[pl.ds(r8, _PCH // 8)])
            return i

        lax.fori_loop(0, -(-_NPCH // _NW), chunk, 0)

        @pl.when(wid == 0)
        def _():
            start = pl.multiple_of(jnp.int32((_V // 8) * 8), 8)
            pltpu.sync_copy(table_hbm.at[pl.ds(start, 8)],
                            in_v.at[pl.ds(0, 8)])
            for c8 in range(8):
                out_v[0, pl.ds(c8 * _HD, _HD)] = in_v[c8, :]
            tr8 = pl.multiple_of(jnp.int32(_V // 64 * 8), 8)
            pltpu.sync_copy(out_v.at[pl.ds(0, 8)],
                            out_hbm.at[pl.ds(tr8, 8)])

    return k(table)


def _pack_and_gather(ids1, table):
    cmp = _sc_pack(table).reshape(_VP, _HD)
    return _sc_gather(ids1, cmp)


def _sc_gather(ids1, table):
    mesh = plsc.VectorSubcoreMesh(core_axis_name="c", subcore_axis_name="s")

    @functools.partial(
        pl.kernel,
        out_type=jax.ShapeDtypeStruct((_NIDX, _HD), jnp.float32),
        mesh=mesh,
        scratch_types=[
            pltpu.VMEM((_PERW,), jnp.int32),
            pltpu.VMEM((_PERW, _HD), jnp.float32),
            pltpu.SemaphoreType.DMA,
        ],
        compiler_params=pltpu.CompilerParams(use_tc_tiling_on_sc=False),
    )
    def k(ids_hbm, table_hbm, out_hbm, idx_v, rows_v, sem):
        wid = lax.axis_index("s") * 2 + lax.axis_index("c")
        base = wid * _PERW
        pltpu.sync_copy(ids_hbm.at[pl.ds(base, _PERW)], idx_v)

        def g_body(j, _):
            pltpu.async_copy(table_hbm.at[idx_v.at[pl.ds(j * _CHUNK, _CHUNK)]],
                             rows_v.at[pl.ds(j * _CHUNK, _CHUNK)], sem)
            return 0

        lax.fori_loop(0, _NCHUNK, g_body, 0)
        pltpu.make_async_copy(out_hbm.at[pl.ds(base, _PERW)], rows_v, sem).wait()
        pltpu.sync_copy(rows_v, out_hbm.at[pl.ds(base, _PERW)])

    return k(ids1, table)


_BLK = 512
_GD = _G * _D
_RSQD = float(1.0 / np.sqrt(_D))


def _tc_body(emb_ref, hid_ref, wk_ref, wv_ref, bk_ref, bv_ref, qks_ref,
             cs_ref, ck_ref, ones_ref, out_ref, carry_ref):
    @pl.when(pl.program_id(1) == 0)
    def _():
        carry_ref[...] = jnp.zeros_like(carry_ref)

    emb = emb_ref[0]
    ones = ones_ref[...]
    value = jnp.dot(emb, wv_ref[...], preferred_element_type=jnp.float32) + bv_ref[...]
    mv = jnp.dot(value * value, ones,
                 preferred_element_type=jnp.float32)[:, :1] * (1.0 / _D)
    ck = ck_ref[...]
    for g in range(_G):
        hg = hid_ref[0, :, g, :]
        kg = jnp.dot(emb, wk_ref[:, g, :],
                     preferred_element_type=jnp.float32) + bk_ref[g]
        hh = jnp.dot(hg * hg, ones, preferred_element_type=jnp.float32)[:, :1]
        kk = jnp.dot(kg * kg, ones, preferred_element_type=jnp.float32)[:, :1]
        hk = jnp.dot((hg * qks_ref[g]) * kg, ones,
                     preferred_element_type=jnp.float32)[:, :1]
        rh = lax.rsqrt(hh * (1.0 / _D) + _EPS)
        rk = lax.rsqrt(kk * (1.0 / _D) + _EPS)
        gate = jax.nn.sigmoid(hk * rh * rk * _RSQD)
        scale = gate * lax.rsqrt(gate * gate * mv + _EPS)
        xng = (value * cs_ref[g]) * scale
        win = jnp.concatenate([carry_ref[g], xng], axis=0)
        y = xng * ck[_K - 1, g]
        for j in range(_K - 1):
            y = y + win[j * _DIL: j * _DIL + _BLK, :] * ck[j, g]
        out_ref[0, :, g, :] = y * jax.nn.sigmoid(y)
        carry_ref[g] = xng[_BLK - _PAD:, :]


def _tc_fused(emb_flat, hid4, wk3, wv2, bk, bv2, qks, cs, ck3, ones):
    grid = (_B, _S // _BLK)
    return pl.pallas_call(
        _tc_body,
        grid=grid,
        in_specs=[
            pl.BlockSpec((1, _BLK, _E), lambda b, s: (b, s, 0)),
            pl.BlockSpec((1, _BLK, _G, _D), lambda b, s: (b, s, 0, 0)),
            pl.BlockSpec((_E, _G, _D), lambda b, s: (0, 0, 0)),
            pl.BlockSpec((_E, _D), lambda b, s: (0, 0)),
            pl.BlockSpec((_G, _D), lambda b, s: (0, 0)),
            pl.BlockSpec((1, _D), lambda b, s: (0, 0)),
            pl.BlockSpec((_G, _D), lambda b, s: (0, 0)),
            pl.BlockSpec((_G, _D), lambda b, s: (0, 0)),
            pl.BlockSpec((_K, _G, _D), lambda b, s: (0, 0, 0)),
            pl.BlockSpec((_D, 128), lambda b, s: (0, 0)),
        ],
        out_specs=pl.BlockSpec((1, _BLK, _G, _D), lambda b, s: (b, s, 0, 0)),
        out_shape=jax.ShapeDtypeStruct((_B, _S, _G, _D), jnp.float32),
        scratch_shapes=[pltpu.VMEM((_G, _PAD, _D), jnp.float32)],
    )(emb_flat, hid4, wk3, wv2, bk, bv2, qks, cs, ck3, ones)


def kernel(hidden_states, hash_input_ids, emb_table, W_k, b_k, W_v, b_v,
           q_scale, k_scale, conv_norm_scale, conv_kernel):
    ids2 = hash_input_ids.reshape(_B * _S, _H)
    sid1 = _tc_prep(ids2).reshape(_NIDX)
    emb_flat = _pack_and_gather(sid1, emb_table).reshape(_B, _S, _E)
    bv2 = b_v.reshape(1, _D)
    ck3 = conv_kernel.reshape(_K, _G, _D)
    qks = q_scale * k_scale
    ones = jnp.ones((_D, 128), dtype=jnp.float32)
    return _tc_fused(emb_flat, hidden_states, W_k, W_v, b_k, bv2,
                     qks, conv_norm_scale, ck3, ones)


# speedup vs baseline: 1.2287x; 1.2287x over previous
import functools

import jax
import jax.numpy as jnp
import numpy as np
from jax import lax
from jax.experimental import pallas as pl
from jax.experimental.pallas import tpu as pltpu
from jax.experimental.pallas import tpu_sc as plsc

_VOCAB_SIZES = [100003, 100019, 100043, 100057, 100069, 100103, 100109,
                100129, 100151, 100153, 100169, 100183, 100189, 100193,
                100207, 100213]
_OFFSETS = np.concatenate([[0], np.cumsum(_VOCAB_SIZES)[:-1]]).astype(np.int32)
_B, _S, _G, _D = 4, 2048, 2, 1024
_H = 16
_HD = 16
_E = _H * _HD
_K = 4
_DIL = 3
_PAD = (_K - 1) * _DIL
_EPS = 1e-6

_NW = 32
_NIDX = _B * _S * _H
_PERW = _NIDX // _NW
_CHUNK = 128
_NCHUNK = _PERW // _CHUNK


_V = sum(_VOCAB_SIZES)
_CV = 8192
_NCV = -(-_V // _CV)
_VP = _NCV * _CV


def _tc_prep(ids2):
    def body(i_ref, offs_ref, o_ref):
        x = i_ref[...] + offs_ref[...][0:1, :]
        x3 = x.reshape(_NIDX // 128, 8, _H)
        o_ref[...] = jnp.concatenate([x3[:, c, :] for c in range(8)], axis=1)

    offs8 = jnp.tile(jnp.asarray(_OFFSETS, dtype=jnp.int32)[None, :], (8, 1))
    return pl.pallas_call(
        body,
        in_specs=[pl.BlockSpec((_B * _S, _H), lambda: (0, 0)),
                  pl.BlockSpec((8, _H), lambda: (0, 0))],
        out_specs=pl.BlockSpec((_NIDX // 128, 128), lambda: (0, 0)),
        out_shape=jax.ShapeDtypeStruct((_NIDX // 128, 128), jnp.int32),
    )(ids2, offs8)


_PCH = 256
_NPCH = -(-(_V - 8) // _PCH)
_PAL = (_V - _PCH) // 8 * 8
_NITER = -(-_NPCH // _NW)


def _sc_pack(table):
    mesh = plsc.VectorSubcoreMesh(core_axis_name="c", subcore_axis_name="s")

    @functools.partial(
        pl.kernel,
        out_type=jax.ShapeDtypeStruct((_VP // 8, 128), jnp.float32),
        mesh=mesh,
        scratch_types=[
            pltpu.VMEM((2, _PCH, _HD), jnp.float32),
            pltpu.VMEM((2, _PCH // 8, 128), jnp.float32),
            pltpu.SemaphoreType.DMA,
            pltpu.SemaphoreType.DMA,
        ],
    )
    def k(table_hbm, out_hbm, in_v, out_v, isem, osem):
        wid = lax.axis_index("s") * 2 + lax.axis_index("c")

        def row_of(i):
            c = i * _NW + wid
            return pl.multiple_of(jnp.minimum(c * _PCH, _PAL), 64)

        pltpu.async_copy(table_hbm.at[pl.ds(row_of(0), _PCH)],
                         in_v.at[0], isem)

        def chunk(i, _):
            buf = lax.rem(i, 2)

            @pl.when(i + 1 < _NITER)
            def _():
                pltpu.async_copy(
                    table_hbm.at[pl.ds(row_of(i + 1), _PCH)],
                    in_v.at[lax.rem(i + 1, 2)], isem)

            pltpu.make_async_copy(table_hbm.at[pl.ds(0, _PCH)],
                                  in_v.at[buf], isem).wait()

            @pl.when(i >= 2)
            def _():
                pltpu.make_async_copy(
                    out_hbm.at[pl.ds(0, _PCH // 8)],
                    out_v.at[buf], osem).wait()

            def cb(j, _):
                for c8 in range(8):
                    out_v[buf, j, pl.ds(c8 * _HD, _HD)] = \
                        in_v[buf, 8 * j + c8, :]
                return j

            lax.fori_loop(0, _PCH // 8, cb, 0)
            r8 = pl.multiple_of(row_of(i) // 8, 8)
            pltpu.async_copy(out_v.at[buf],
                             out_hbm.at[pl.ds(r8, _PCH // 8)], osem)
            return i

        lax.fori_loop(0, _NITER, chunk, 0)
        pltpu.make_async_copy(out_hbm.at[pl.ds(0, _PCH // 8)],
                              out_v.at[0], osem).wait()
        pltpu.make_async_copy(out_hbm.at[pl.ds(0, _PCH // 8)],
                              out_v.at[1], osem).wait()

        @pl.when(wid == 0)
        def _():
            start = pl.multiple_of(jnp.int32((_V // 8) * 8), 8)
            pltpu.sync_copy(table_hbm.at[pl.ds(start, 8)],
                            in_v.at[0, pl.ds(0, 8)])
            for c8 in range(8):
                out_v[0, 0, pl.ds(c8 * _HD, _HD)] = in_v[0, c8, :]
            tr8 = pl.multiple_of(jnp.int32(_V // 64 * 8), 8)
            pltpu.sync_copy(out_v.at[0, pl.ds(0, 8)],
                            out_hbm.at[pl.ds(tr8, 8)])

    return k(table)


def _pack_and_gather(ids1, table):
    cmp = _sc_pack(table).reshape(_VP, _HD)
    return _sc_gather(ids1, cmp)


def _sc_gather(ids1, table):
    mesh = plsc.VectorSubcoreMesh(core_axis_name="c", subcore_axis_name="s")

    @functools.partial(
        pl.kernel,
        out_type=jax.ShapeDtypeStruct((_NIDX, _HD), jnp.float32),
        mesh=mesh,
        scratch_types=[
            pltpu.VMEM((_PERW,), jnp.int32),
            pltpu.VMEM((_PERW, _HD), jnp.float32),
            pltpu.SemaphoreType.DMA,
        ],
        compiler_params=pltpu.CompilerParams(use_tc_tiling_on_sc=False),
    )
    def k(ids_hbm, table_hbm, out_hbm, idx_v, rows_v, sem):
        wid = lax.axis_index("s") * 2 + lax.axis_index("c")
        base = wid * _PERW
        pltpu.sync_copy(ids_hbm.at[pl.ds(base, _PERW)], idx_v)

        def g_body(j, _):
            pltpu.async_copy(table_hbm.at[idx_v.at[pl.ds(j * _CHUNK, _CHUNK)]],
                             rows_v.at[pl.ds(j * _CHUNK, _CHUNK)], sem)
            return 0

        lax.fori_loop(0, _NCHUNK, g_body, 0)
        pltpu.make_async_copy(out_hbm.at[pl.ds(base, _PERW)], rows_v, sem).wait()
        pltpu.sync_copy(rows_v, out_hbm.at[pl.ds(base, _PERW)])

    return k(ids1, table)


_BLK = 512
_GD = _G * _D
_RSQD = float(1.0 / np.sqrt(_D))


def _tc_body(emb_ref, hid_ref, wk_ref, wv_ref, bk_ref, bv_ref, qks_ref,
             cs_ref, ck_ref, ones_ref, out_ref, carry_ref):
    @pl.when(pl.program_id(1) == 0)
    def _():
        carry_ref[...] = jnp.zeros_like(carry_ref)

    emb = emb_ref[0]
    ones = ones_ref[...]
    value = jnp.dot(emb, wv_ref[...], preferred_element_type=jnp.float32) + bv_ref[...]
    mv = jnp.dot(value * value, ones,
                 preferred_element_type=jnp.float32)[:, :1] * (1.0 / _D)
    ck = ck_ref[...]
    for g in range(_G):
        hg = hid_ref[0, :, g, :]
        kg = jnp.dot(emb, wk_ref[:, g, :],
                     preferred_element_type=jnp.float32) + bk_ref[g]
        hh = jnp.dot(hg * hg, ones, preferred_element_type=jnp.float32)[:, :1]
        kk = jnp.dot(kg * kg, ones, preferred_element_type=jnp.float32)[:, :1]
        hk = jnp.dot((hg * qks_ref[g]) * kg, ones,
                     preferred_element_type=jnp.float32)[:, :1]
        rh = lax.rsqrt(hh * (1.0 / _D) + _EPS)
        rk = lax.rsqrt(kk * (1.0 / _D) + _EPS)
        gate = jax.nn.sigmoid(hk * rh * rk * _RSQD)
        scale = gate * lax.rsqrt(gate * gate * mv + _EPS)
        xng = (value * cs_ref[g]) * scale
        win = jnp.concatenate([carry_ref[g], xng], axis=0)
        y = xng * ck[_K - 1, g]
        for j in range(_K - 1):
            y = y + win[j * _DIL: j * _DIL + _BLK, :] * ck[j, g]
        out_ref[0, :, g, :] = y * jax.nn.sigmoid(y)
        carry_ref[g] = xng[_BLK - _PAD:, :]


def _tc_fused(emb_flat, hid4, wk3, wv2, bk, bv2, qks, cs, ck3, ones):
    grid = (_B, _S // _BLK)
    return pl.pallas_call(
        _tc_body,
        grid=grid,
        in_specs=[
            pl.BlockSpec((1, _BLK, _E), lambda b, s: (b, s, 0)),
            pl.BlockSpec((1, _BLK, _G, _D), lambda b, s: (b, s, 0, 0)),
            pl.BlockSpec((_E, _G, _D), lambda b, s: (0, 0, 0)),
            pl.BlockSpec((_E, _D), lambda b, s: (0, 0)),
            pl.BlockSpec((_G, _D), lambda b, s: (0, 0)),
            pl.BlockSpec((1, _D), lambda b, s: (0, 0)),
            pl.BlockSpec((_G, _D), lambda b, s: (0, 0)),
            pl.BlockSpec((_G, _D), lambda b, s: (0, 0)),
            pl.BlockSpec((_K, _G, _D), lambda b, s: (0, 0, 0)),
            pl.BlockSpec((_D, 128), lambda b, s: (0, 0)),
        ],
        out_specs=pl.BlockSpec((1, _BLK, _G, _D), lambda b, s: (b, s, 0, 0)),
        out_shape=jax.ShapeDtypeStruct((_B, _S, _G, _D), jnp.float32),
        scratch_shapes=[pltpu.VMEM((_G, _PAD, _D), jnp.float32)],
    )(emb_flat, hid4, wk3, wv2, bk, bv2, qks, cs, ck3, ones)


def kernel(hidden_states, hash_input_ids, emb_table, W_k, b_k, W_v, b_v,
           q_scale, k_scale, conv_norm_scale, conv_kernel):
    ids2 = hash_input_ids.reshape(_B * _S, _H)
    sid1 = _tc_prep(ids2).reshape(_NIDX)
    emb_flat = _pack_and_gather(sid1, emb_table).reshape(_B, _S, _E)
    bv2 = b_v.reshape(1, _D)
    ck3 = conv_kernel.reshape(_K, _G, _D)
    qks = q_scale * k_scale
    ones = jnp.ones((_D, 128), dtype=jnp.float32)
    return _tc_fused(emb_flat, hidden_states, W_k, W_v, b_k, bv2,
                     qks, conv_norm_scale, ck3, ones)


# speedup vs baseline: 1.4472x; 1.1779x over previous
import functools

import jax
import jax.numpy as jnp
import numpy as np
from jax import lax
from jax.experimental import pallas as pl
from jax.experimental.pallas import tpu as pltpu
from jax.experimental.pallas import tpu_sc as plsc

_VOCAB_SIZES = [100003, 100019, 100043, 100057, 100069, 100103, 100109,
                100129, 100151, 100153, 100169, 100183, 100189, 100193,
                100207, 100213]
_OFFSETS = np.concatenate([[0], np.cumsum(_VOCAB_SIZES)[:-1]]).astype(np.int32)
_B, _S, _G, _D = 4, 2048, 2, 1024
_H = 16
_HD = 16
_E = _H * _HD
_K = 4
_DIL = 3
_PAD = (_K - 1) * _DIL
_EPS = 1e-6

_NW = 32
_NIDX = _B * _S * _H
_PERW = _NIDX // _NW
_CHUNK = 128
_NCHUNK = _PERW // _CHUNK


_V = sum(_VOCAB_SIZES)
_CV = 8192
_NCV = -(-_V // _CV)
_VP = _NCV * _CV


def _tc_prep(ids2):
    def body(i_ref, offs_ref, o_ref):
        x = i_ref[...] + offs_ref[...][0:1, :]
        x3 = x.reshape(_NIDX // 128, 8, _H)
        o_ref[...] = jnp.concatenate([x3[:, c, :] for c in range(8)], axis=1)

    offs8 = jnp.tile(jnp.asarray(_OFFSETS, dtype=jnp.int32)[None, :], (8, 1))
    return pl.pallas_call(
        body,
        in_specs=[pl.BlockSpec((_B * _S, _H), lambda: (0, 0)),
                  pl.BlockSpec((8, _H), lambda: (0, 0))],
        out_specs=pl.BlockSpec((_NIDX // 128, 128), lambda: (0, 0)),
        out_shape=jax.ShapeDtypeStruct((_NIDX // 128, 128), jnp.int32),
    )(ids2, offs8)


def _sc_gather(ids1, table):
    mesh = plsc.VectorSubcoreMesh(core_axis_name="c", subcore_axis_name="s")

    @functools.partial(
        pl.kernel,
        out_type=jax.ShapeDtypeStruct((_NIDX, _HD), jnp.float32),
        mesh=mesh,
        scratch_types=[
            pltpu.VMEM((_PERW,), jnp.int32),
            pltpu.VMEM((_PERW, _HD), jnp.float32),
            pltpu.SemaphoreType.DMA,
        ],
        compiler_params=pltpu.CompilerParams(use_tc_tiling_on_sc=False),
    )
    def k(ids_hbm, table_hbm, out_hbm, idx_v, rows_v, sem):
        wid = lax.axis_index("s") * 2 + lax.axis_index("c")
        base = wid * _PERW
        pltpu.sync_copy(ids_hbm.at[pl.ds(base, _PERW)], idx_v)

        def g_body(j, _):
            pltpu.async_copy(table_hbm.at[idx_v.at[pl.ds(j * _CHUNK, _CHUNK)]],
                             rows_v.at[pl.ds(j * _CHUNK, _CHUNK)], sem)
            return 0

        lax.fori_loop(0, _NCHUNK, g_body, 0)
        pltpu.make_async_copy(out_hbm.at[pl.ds(base, _PERW)], rows_v, sem).wait()
        pltpu.sync_copy(rows_v, out_hbm.at[pl.ds(base, _PERW)])

    return k(ids1, table)


_BLK = 512
_GD = _G * _D
_RSQD = float(1.0 / np.sqrt(_D))


def _tc_body(emb_ref, hid_ref, wk_ref, wv_ref, bk_ref, bv_ref, qks_ref,
             cs_ref, ck_ref, ones_ref, out_ref, carry_ref):
    @pl.when(pl.program_id(1) == 0)
    def _():
        carry_ref[...] = jnp.zeros_like(carry_ref)

    emb = emb_ref[0]
    ones = ones_ref[...]
    value = jnp.dot(emb, wv_ref[...], preferred_element_type=jnp.float32) + bv_ref[...]
    mv = jnp.dot(value * value, ones,
                 preferred_element_type=jnp.float32)[:, :1] * (1.0 / _D)
    ck = ck_ref[...]
    for g in range(_G):
        hg = hid_ref[0, :, g, :]
        kg = jnp.dot(emb, wk_ref[:, g, :],
                     preferred_element_type=jnp.float32) + bk_ref[g]
        hh = jnp.dot(hg * hg, ones, preferred_element_type=jnp.float32)[:, :1]
        kk = jnp.dot(kg * kg, ones, preferred_element_type=jnp.float32)[:, :1]
        hk = jnp.dot((hg * qks_ref[g]) * kg, ones,
                     preferred_element_type=jnp.float32)[:, :1]
        rh = lax.rsqrt(hh * (1.0 / _D) + _EPS)
        rk = lax.rsqrt(kk * (1.0 / _D) + _EPS)
        gate = jax.nn.sigmoid(hk * rh * rk * _RSQD)
        scale = gate * lax.rsqrt(gate * gate * mv + _EPS)
        xng = (value * cs_ref[g]) * scale
        win = jnp.concatenate([carry_ref[g], xng], axis=0)
        y = xng * ck[_K - 1, g]
        for j in range(_K - 1):
            y = y + win[j * _DIL: j * _DIL + _BLK, :] * ck[j, g]
        out_ref[0, :, g, :] = y * jax.nn.sigmoid(y)
        carry_ref[g] = xng[_BLK - _PAD:, :]


def _tc_fused(emb_flat, hid4, wk3, wv2, bk, bv2, qks, cs, ck3, ones):
    grid = (_B, _S // _BLK)
    return pl.pallas_call(
        _tc_body,
        grid=grid,
        in_specs=[
            pl.BlockSpec((1, _BLK, _E), lambda b, s: (b, s, 0)),
            pl.BlockSpec((1, _BLK, _G, _D), lambda b, s: (b, s, 0, 0)),
            pl.BlockSpec((_E, _G, _D), lambda b, s: (0, 0, 0)),
            pl.BlockSpec((_E, _D), lambda b, s: (0, 0)),
            pl.BlockSpec((_G, _D), lambda b, s: (0, 0)),
            pl.BlockSpec((1, _D), lambda b, s: (0, 0)),
            pl.BlockSpec((_G, _D), lambda b, s: (0, 0)),
            pl.BlockSpec((_G, _D), lambda b, s: (0, 0)),
            pl.BlockSpec((_K, _G, _D), lambda b, s: (0, 0, 0)),
            pl.BlockSpec((_D, 128), lambda b, s: (0, 0)),
        ],
        out_specs=pl.BlockSpec((1, _BLK, _G, _D), lambda b, s: (b, s, 0, 0)),
        out_shape=jax.ShapeDtypeStruct((_B, _S, _G, _D), jnp.float32),
        scratch_shapes=[pltpu.VMEM((_G, _PAD, _D), jnp.float32)],
    )(emb_flat, hid4, wk3, wv2, bk, bv2, qks, cs, ck3, ones)


def kernel(hidden_states, hash_input_ids, emb_table, W_k, b_k, W_v, b_v,
           q_scale, k_scale, conv_norm_scale, conv_kernel):
    ids2 = hash_input_ids.reshape(_B * _S, _H)
    sid1 = _tc_prep(ids2).reshape(_NIDX)
    emb_flat = _sc_gather(sid1, emb_table).reshape(_B, _S, _E)
    bv2 = b_v.reshape(1, _D)
    ck3 = conv_kernel.reshape(_K, _G, _D)
    qks = q_scale * k_scale
    ones = jnp.ones((_D, 128), dtype=jnp.float32)
    return _tc_fused(emb_flat, hidden_states, W_k, W_v, b_k, bv2,
                     qks, conv_norm_scale, ck3, ones)
